# baseline (speedup 1.0000x reference)
import jax
import jax.numpy as jnp
from jax.experimental import pallas as pl
from jax.experimental.pallas import tpu as pltpu

B, D, S, T, H, HOR = 4096, 256, 16, 220, 64, 20
TP = T - HOR
TP1, TP2 = 104, 96
TAU = 1.0
B2 = B // 2
BQ = B2 // 2
K = 2 * H + 2 * S
G = 8 * H
U = 8


def _fold(a):
    return a.reshape(B2, 2 * a.shape[-1])


def _make_body(nt, do_finish):
    def body(xs_ref, xf_ref, Wg_ref, bg_ref, Wp2_ref, Wfh2_ref, Wfx2_ref,
             bfc2_ref, gn2_ref, hxA0_ref, cA0_ref, hxB0_ref, cB0_ref,
             final_ref, preds_ref, hxA1_ref, cA1_ref, hxB1_ref, cB1_ref,
             hxA_ref, cA_ref, hxB_ref, cB_ref):
        t = pl.program_id(0)

        @pl.when(t == 0)
        def _init():
            hxA_ref[...] = hxA0_ref[...]
            cA_ref[...] = cA0_ref[...]
            hxB_ref[...] = hxB0_ref[...]
            cB_ref[...] = cB0_ref[...]

        def step(u, r, hx_ref, c_ref):
            rows = pl.ds(r * BQ, BQ)
            hx_ref[:, 2 * H:] = xs_ref[u, rows]
            gates = jnp.dot(hx_ref[...], Wg_ref[...],
                            preferred_element_type=jnp.float32) + bg_ref[...]
            i = jnp.tanh(gates[:, 0 * 2 * H:1 * 2 * H]) * 0.5 + 0.5
            f = jnp.tanh(gates[:, 1 * 2 * H:2 * 2 * H]) * 0.5 + 0.5
            g = jnp.tanh(gates[:, 2 * 2 * H:3 * 2 * H])
            o = jnp.tanh(gates[:, 3 * 2 * H:4 * 2 * H]) * 0.5 + 0.5
            c = f * c_ref[...] + i * g
            h = o * jnp.tanh(c)
            c_ref[...] = c
            hx_ref[:, :2 * H] = h
            return h

        for u in range(U):
            hA = step(u, 0, hxA_ref, cA_ref)
            hB = step(u, 1, hxB_ref, cB_ref)

        @pl.when(t == nt - 1)
        def _emit_state():
            hxA1_ref[...] = hxA_ref[...]
            cA1_ref[...] = cA_ref[...]
            hxB1_ref[...] = hxB_ref[...]
            cB1_ref[...] = cB_ref[...]

        if do_finish:
            @pl.when(t == nt - 1)
            def _finish():
                for r, h in ((0, hA), (1, hB)):
                    rows = pl.ds(r * BQ, BQ)
                    xf = xf_ref[rows, :]
                    preds2 = jnp.dot(xf, Wp2_ref[...],
                                     preferred_element_type=jnp.float32)
                    preds_ref[rows, :] = preds2
                    out2 = (jnp.dot(h, Wfh2_ref[...],
                                    preferred_element_type=jnp.float32)
                            + jnp.dot(xf, Wfx2_ref[...],
                                      preferred_element_type=jnp.float32)
                            + bfc2_ref[...])
                    logits2 = (out2 + gn2_ref[rows, :]) * (1.0 / TAU)
                    lo, hi = logits2[:, :S], logits2[:, S:]
                    plo, phi = preds2[:, :S], preds2[:, S:]
                    elo = jnp.exp(lo - jnp.max(lo, axis=-1, keepdims=True))
                    ehi = jnp.exp(hi - jnp.max(hi, axis=-1, keepdims=True))
                    flo = jnp.sum(plo * elo, axis=-1, keepdims=True) / \
                        jnp.sum(elo, axis=-1, keepdims=True)
                    fhi = jnp.sum(phi * ehi, axis=-1, keepdims=True) / \
                        jnp.sum(ehi, axis=-1, keepdims=True)
                    final_ref[rows, :] = jnp.concatenate([flo, fhi],
                                                         axis=-1)
        else:
            del final_ref, preds_ref

    return body


def _run_chunk(xs2, state, consts, nt, do_finish):
    xf, Wg, bg, Wp2, Wfh2, Wfx2, bfc2, gn2 = consts
    cmap = lambda t: (0, 0)
    return pl.pallas_call(
        _make_body(nt // U, do_finish),
        grid=(nt // U,),
        in_specs=[
            pl.BlockSpec((U, B2, 2 * S), lambda t: (t, 0, 0)),
            pl.BlockSpec((B2, 2 * D), cmap),
            pl.BlockSpec((K, G), cmap),
            pl.BlockSpec((1, G), cmap),
            pl.BlockSpec((2 * D, 2 * S), cmap),
            pl.BlockSpec((2 * H, 2 * S), cmap),
            pl.BlockSpec((2 * D, 2 * S), cmap),
            pl.BlockSpec((1, 2 * S), cmap),
            pl.BlockSpec((B2, 2 * S), cmap),
            pl.BlockSpec((BQ, K), cmap),
            pl.BlockSpec((BQ, 2 * H), cmap),
            pl.BlockSpec((BQ, K), cmap),
            pl.BlockSpec((BQ, 2 * H), cmap),
        ],
        out_specs=[
            pl.BlockSpec((B2, 2), cmap),
            pl.BlockSpec((B2, 2 * S), cmap),
            pl.BlockSpec((BQ, K), cmap),
            pl.BlockSpec((BQ, 2 * H), cmap),
            pl.BlockSpec((BQ, K), cmap),
            pl.BlockSpec((BQ, 2 * H), cmap),
        ],
        out_shape=[
            jax.ShapeDtypeStruct((B2, 2), jnp.float32),
            jax.ShapeDtypeStruct((B2, 2 * S), jnp.float32),
            jax.ShapeDtypeStruct((BQ, K), jnp.float32),
            jax.ShapeDtypeStruct((BQ, 2 * H), jnp.float32),
            jax.ShapeDtypeStruct((BQ, K), jnp.float32),
            jax.ShapeDtypeStruct((BQ, 2 * H), jnp.float32),
        ],
        scratch_shapes=[
            pltpu.VMEM((BQ, K), jnp.float32),
            pltpu.VMEM((BQ, 2 * H), jnp.float32),
            pltpu.VMEM((BQ, K), jnp.float32),
            pltpu.VMEM((BQ, 2 * H), jnp.float32),
        ],
    )(xs2, xf, Wg, bg, Wp2, Wfh2, Wfx2, bfc2, gn2, *state)


def _block_diag2(w):
    r, c = w.shape
    z = jnp.zeros((r, c), w.dtype)
    return jnp.block([[w, z], [z, w]])


def _fold_hist(hist_loss, lo, hi):
    return jnp.transpose(hist_loss.reshape(B2, 2, T, S)[:, :, lo:hi],
                         (2, 0, 1, 3)).reshape(hi - lo, B2, 2 * S)


@jax.jit
def kernel(x, hist_loss, Wp, bp, W_ih, W_hh, b_ih, b_hh, Wfc, bfc):
    xs2a = _fold_hist(hist_loss, 0, TP1)
    xs2b = _fold_hist(hist_loss, TP1, TP)

    WhT = W_hh.T
    WxT = W_ih.T
    b = b_ih + b_hh
    Wg = jnp.zeros((K, G), jnp.float32)
    bg = jnp.zeros((G,), jnp.float32)
    for q in range(4):
        s = 1.0 if q == 2 else 0.5
        wh = WhT[:, q * H:(q + 1) * H] * s
        wx = WxT[:, q * H:(q + 1) * H] * s
        Wg = Wg.at[0:H, q * 2 * H:q * 2 * H + H].set(wh)
        Wg = Wg.at[H:2 * H, q * 2 * H + H:(q + 1) * 2 * H].set(wh)
        Wg = Wg.at[2 * H:2 * H + S, q * 2 * H:q * 2 * H + H].set(wx)
        Wg = Wg.at[2 * H + S:K, q * 2 * H + H:(q + 1) * 2 * H].set(wx)
        bg = bg.at[q * 2 * H:q * 2 * H + H].set(b[q * H:(q + 1) * H] * s)
        bg = bg.at[q * 2 * H + H:(q + 1) * 2 * H].set(b[q * H:(q + 1) * H] * s)

    xf = _fold(x)
    Wp2 = _block_diag2(Wp.T)
    Wfh2 = _block_diag2(Wfc[:, :H].T)
    Wfx2 = _block_diag2(Wfc[:, H:].T)
    bfc2 = jnp.tile(bfc, 2)[None, :]
    gn2 = _fold(jax.random.gumbel(jax.random.key(42), (B, S),
                                  dtype=jnp.float32))
    consts = (xf, Wg, bg[None, :], Wp2, Wfh2, Wfx2, bfc2, gn2)

    z_hx = jnp.zeros((BQ, K), jnp.float32)
    z_c = jnp.zeros((BQ, 2 * H), jnp.float32)
    out1 = _run_chunk(xs2a, (z_hx, z_c, z_hx, z_c), consts, TP1, False)
    out2 = _run_chunk(xs2b, tuple(out1[2:]), consts, TP2, True)

    final_pred = out2[0].reshape(B, 1)
    preds = out2[1].reshape(B, S)
    return (final_pred, preds)


# speedup vs baseline: 1.1153x; 1.1153x over previous
import jax
import jax.numpy as jnp
from jax.experimental import pallas as pl
from jax.experimental.pallas import tpu as pltpu

B, D, S, T, H, HOR = 4096, 256, 16, 220, 64, 20
TP = T - HOR
TAU = 1.0
B2 = B // 2
BQ = B2 // 2
K = 2 * H + 2 * S
G = 8 * H
U = 8


def _fold(a):
    return a.reshape(B2, 2 * a.shape[-1])


def _unfold(a2):
    return a2.reshape(B, a2.shape[-1] // 2)


def _lstm_router_kernel(xs_ref, xf_ref, Wg_ref, bg_ref, Wp2_ref, Wfh2_ref,
                        Wfx2_ref, bfc2_ref, gn2_ref, final_ref, preds_ref,
                        hxA_ref, cA_ref, hxB_ref, cB_ref):
    t = pl.program_id(0)

    @pl.when(t == 0)
    def _init():
        hxA_ref[...] = jnp.zeros_like(hxA_ref)
        cA_ref[...] = jnp.zeros_like(cA_ref)
        hxB_ref[...] = jnp.zeros_like(hxB_ref)
        cB_ref[...] = jnp.zeros_like(cB_ref)

    def step(u, r, hx_ref, c_ref):
        rows = pl.ds(r * BQ, BQ)
        hx_ref[:, 2 * H:] = xs_ref[u, rows]
        gates = jnp.dot(hx_ref[...], Wg_ref[...],
                        preferred_element_type=jnp.float32) + bg_ref[...]
        i = jnp.tanh(gates[:, 0 * 2 * H:1 * 2 * H]) * 0.5 + 0.5
        f = jnp.tanh(gates[:, 1 * 2 * H:2 * 2 * H]) * 0.5 + 0.5
        g = jnp.tanh(gates[:, 2 * 2 * H:3 * 2 * H])
        o = jnp.tanh(gates[:, 3 * 2 * H:4 * 2 * H]) * 0.5 + 0.5
        c = f * c_ref[...] + i * g
        h = o * jnp.tanh(c)
        c_ref[...] = c
        hx_ref[:, :2 * H] = h
        return h

    for u in range(U):
        hA = step(u, 0, hxA_ref, cA_ref)
        hB = step(u, 1, hxB_ref, cB_ref)

    @pl.when(t == TP // U - 1)
    def _finish():
        for r, h in ((0, hA), (1, hB)):
            rows = pl.ds(r * BQ, BQ)
            xf = xf_ref[rows, :]
            preds2 = jnp.dot(xf, Wp2_ref[...],
                             preferred_element_type=jnp.float32)
            preds_ref[rows, :] = preds2
            out2 = (jnp.dot(h, Wfh2_ref[...],
                            preferred_element_type=jnp.float32)
                    + jnp.dot(xf, Wfx2_ref[...],
                              preferred_element_type=jnp.float32)
                    + bfc2_ref[...])
            logits2 = (out2 + gn2_ref[rows, :]) * (1.0 / TAU)
            lo, hi = logits2[:, :S], logits2[:, S:]
            plo, phi = preds2[:, :S], preds2[:, S:]
            elo = jnp.exp(lo - jnp.max(lo, axis=-1, keepdims=True))
            ehi = jnp.exp(hi - jnp.max(hi, axis=-1, keepdims=True))
            flo = jnp.sum(plo * elo, axis=-1, keepdims=True) / jnp.sum(
                elo, axis=-1, keepdims=True)
            fhi = jnp.sum(phi * ehi, axis=-1, keepdims=True) / jnp.sum(
                ehi, axis=-1, keepdims=True)
            final_ref[rows, :] = jnp.concatenate([flo, fhi], axis=-1)


def _block_diag2(w):
    r, c = w.shape
    z = jnp.zeros((r, c), w.dtype)
    return jnp.block([[w, z], [z, w]])


@jax.jit
def kernel(x, hist_loss, Wp, bp, W_ih, W_hh, b_ih, b_hh, Wfc, bfc):
    xs2 = jnp.transpose(hist_loss.reshape(B2, 2, T, S)[:, :, :TP],
                        (2, 0, 1, 3)).reshape(TP, B2, 2 * S)

    WhT = W_hh.T
    WxT = W_ih.T
    b = b_ih + b_hh
    Wg = jnp.zeros((K, G), jnp.float32)
    bg = jnp.zeros((G,), jnp.float32)
    for q in range(4):
        s = 1.0 if q == 2 else 0.5
        wh = WhT[:, q * H:(q + 1) * H] * s
        wx = WxT[:, q * H:(q + 1) * H] * s
        Wg = Wg.at[0:H, q * 2 * H:q * 2 * H + H].set(wh)
        Wg = Wg.at[H:2 * H, q * 2 * H + H:(q + 1) * 2 * H].set(wh)
        Wg = Wg.at[2 * H:2 * H + S, q * 2 * H:q * 2 * H + H].set(wx)
        Wg = Wg.at[2 * H + S:K, q * 2 * H + H:(q + 1) * 2 * H].set(wx)
        bg = bg.at[q * 2 * H:q * 2 * H + H].set(b[q * H:(q + 1) * H] * s)
        bg = bg.at[q * 2 * H + H:(q + 1) * 2 * H].set(b[q * H:(q + 1) * H] * s)

    xf = _fold(x)
    Wp2 = _block_diag2(Wp.T)
    Wfh2 = _block_diag2(Wfc[:, :H].T)
    Wfx2 = _block_diag2(Wfc[:, H:].T)
    bfc2 = jnp.tile(bfc, 2)[None, :]
    gn2 = _fold(jax.random.gumbel(jax.random.key(42), (B, S),
                                  dtype=jnp.float32))

    final2, preds2 = pl.pallas_call(
        _lstm_router_kernel,
        grid=(TP // U,),
        in_specs=[
            pl.BlockSpec((U, B2, 2 * S), lambda t: (t, 0, 0)),
            pl.BlockSpec((B2, 2 * D), lambda t: (0, 0)),
            pl.BlockSpec((K, G), lambda t: (0, 0)),
            pl.BlockSpec((1, G), lambda t: (0, 0)),
            pl.BlockSpec((2 * D, 2 * S), lambda t: (0, 0)),
            pl.BlockSpec((2 * H, 2 * S), lambda t: (0, 0)),
            pl.BlockSpec((2 * D, 2 * S), lambda t: (0, 0)),
            pl.BlockSpec((1, 2 * S), lambda t: (0, 0)),
            pl.BlockSpec((B2, 2 * S), lambda t: (0, 0)),
        ],
        out_specs=[
            pl.BlockSpec((B2, 2), lambda t: (0, 0)),
            pl.BlockSpec((B2, 2 * S), lambda t: (0, 0)),
        ],
        out_shape=[
            jax.ShapeDtypeStruct((B2, 2), jnp.float32),
            jax.ShapeDtypeStruct((B2, 2 * S), jnp.float32),
        ],
        scratch_shapes=[
            pltpu.VMEM((BQ, K), jnp.float32),
            pltpu.VMEM((BQ, 2 * H), jnp.float32),
            pltpu.VMEM((BQ, K), jnp.float32),
            pltpu.VMEM((BQ, 2 * H), jnp.float32),
        ],
    )(xs2, xf, Wg, bg[None, :], Wp2, Wfh2, Wfx2, bfc2, gn2)

    final_pred = final2.reshape(B, 1)
    preds = _unfold(preds2)
    return (final_pred, preds)
